# Initial kernel scaffold; baseline (speedup 1.0000x reference)
import jax
import jax.numpy as jnp
from jax.experimental import pallas as pl


def kernel(x, lookup_per_prot_emb, seq_sim):
    raise NotImplementedError("write your pallas kernel here")


# speedup vs baseline: 2.8446x; 2.8446x over previous
import jax
import jax.numpy as jnp
from jax.experimental import pallas as pl

K_ROWS = 100000
DIM = 1024
TOP_K = 10
BLK_ROWS = 2000
N_BLK = K_ROWS // BLK_ROWS
_BIG_I32 = 2**30


def _top10_vec(vals, lin):
    lane = jax.lax.broadcasted_iota(jnp.int32, (1, 16), 1)
    out_v = jnp.zeros((1, 16), jnp.float32)
    out_i = jnp.zeros((1, 16), jnp.int32)
    for k in range(TOP_K):
        m = jnp.max(vals)
        idx = jnp.min(jnp.where(vals == m, lin, _BIG_I32))
        out_v = jnp.where(lane == k, m, out_v)
        out_i = jnp.where(lane == k, idx, out_i)
        vals = jnp.where(lin == idx, jnp.float32(-jnp.inf), vals)
    return out_v, out_i


def _body(x_ref, lookup_ref, seq_ref, euc_ref, ev_ref, ei_ref, sv_ref, si_ref):
    i = pl.program_id(0)
    d = lookup_ref[...] - x_ref[...]
    s = jnp.sum(d * d, axis=1)
    euc_ref[pl.ds(i, 1), :] = jnp.sqrt(s).reshape(1, BLK_ROWS)

    @pl.when(i == N_BLK - 1)
    def _():
        row = jax.lax.broadcasted_iota(jnp.int32, (N_BLK, BLK_ROWS), 0)
        col = jax.lax.broadcasted_iota(jnp.int32, (N_BLK, BLK_ROWS), 1)
        lin = row * BLK_ROWS + col
        ev, ei = _top10_vec(euc_ref[...], lin)
        ev_ref[...] = ev
        ei_ref[...] = ei
        sv, si = _top10_vec(seq_ref[...], lin)
        sv_ref[...] = sv
        si_ref[...] = si


def kernel(x, lookup_per_prot_emb, seq_sim):
    x2d = x.reshape(1, DIM)
    seq2d = seq_sim.reshape(N_BLK, BLK_ROWS)
    out = pl.pallas_call(
        _body,
        grid=(N_BLK,),
        in_specs=[
            pl.BlockSpec((1, DIM), lambda i: (0, 0)),
            pl.BlockSpec((BLK_ROWS, DIM), lambda i: (i, 0)),
            pl.BlockSpec((N_BLK, BLK_ROWS), lambda i: (0, 0)),
        ],
        out_specs=[
            pl.BlockSpec((N_BLK, BLK_ROWS), lambda i: (0, 0)),
            pl.BlockSpec((1, 16), lambda i: (0, 0)),
            pl.BlockSpec((1, 16), lambda i: (0, 0)),
            pl.BlockSpec((1, 16), lambda i: (0, 0)),
            pl.BlockSpec((1, 16), lambda i: (0, 0)),
        ],
        out_shape=[
            jax.ShapeDtypeStruct((N_BLK, BLK_ROWS), jnp.float32),
            jax.ShapeDtypeStruct((1, 16), jnp.float32),
            jax.ShapeDtypeStruct((1, 16), jnp.int32),
            jax.ShapeDtypeStruct((1, 16), jnp.float32),
            jax.ShapeDtypeStruct((1, 16), jnp.int32),
        ],
    )(x2d, lookup_per_prot_emb, seq2d)
    euc2d, ev, ei, sv, si = out
    return (euc2d.reshape(K_ROWS), ev[0, :TOP_K], ei[0, :TOP_K],
            sv[0, :TOP_K], si[0, :TOP_K])
